# Initial kernel scaffold; baseline (speedup 1.0000x reference)
import jax
import jax.numpy as jnp
from jax import lax
from jax.experimental import pallas as pl
from jax.experimental.pallas import tpu as pltpu
from jax.experimental.pallas import tpu_sc as plsc

_V = 1_000_000
_D = 32
_B = 16384 * 26

_NC = 2
_NS = 16
_NW = _NC * _NS
_CH = 128
_B_PER_W = _B // _NW
_CHUNKS = _B_PER_W // _CH


def _gather_body(table_hbm, idx_hbm, out_hbm, idx_v, rows_v, sem):
    wid = lax.axis_index("s") * _NC + lax.axis_index("c")
    pltpu.sync_copy(idx_hbm.at[pl.ds(wid * _CHUNKS, _CHUNKS)], idx_v)
    out_base = wid * _B_PER_W

    def body(j, carry):
        pltpu.async_copy(table_hbm.at[idx_v.at[j]], rows_v, sem).wait()
        pltpu.sync_copy(rows_v, out_hbm.at[pl.ds(out_base + j * _CH, _CH)])
        return carry

    lax.fori_loop(0, _CHUNKS, body, 0)


@jax.jit
def _gather(params, idx2d):
    mesh = plsc.VectorSubcoreMesh(core_axis_name="c", subcore_axis_name="s")
    return pl.kernel(
        _gather_body,
        out_type=jax.ShapeDtypeStruct((_B, _D), jnp.float32),
        mesh=mesh,
        scratch_types=[
            pltpu.VMEM((_CHUNKS, _CH), jnp.int32),
            pltpu.VMEM((_CH, _D), jnp.float32),
            pltpu.SemaphoreType.DMA,
        ],
    )(params, idx2d)


def kernel(params, indices):
    idx2d = indices.reshape(-1).astype(jnp.int32).reshape(_NW * _CHUNKS, _CH)
    out = _gather(params, idx2d)
    return out.reshape(indices.shape + (_D,))


# speedup vs baseline: 1.4374x; 1.4374x over previous
import jax
import jax.numpy as jnp
from jax import lax
from jax.experimental import pallas as pl
from jax.experimental.pallas import tpu as pltpu
from jax.experimental.pallas import tpu_sc as plsc

_V = 1_000_000
_D = 32
_B = 16384 * 26

_NC = 2
_NS = 16
_NW = _NC * _NS
_CH = 128
_B_PER_W = _B // _NW
_CHUNKS = _B_PER_W // _CH


def _gather_body(table_hbm, idx_hbm, out_hbm, idx_v, rows_v, sem):
    wid = lax.axis_index("s") * _NC + lax.axis_index("c")
    pltpu.sync_copy(idx_hbm.at[pl.ds(wid * _CHUNKS, _CHUNKS)], idx_v)
    out_base = wid * _B_PER_W

    def body(j, carry):
        pltpu.async_copy(table_hbm.at[idx_v.at[j]], rows_v, sem).wait()
        pltpu.sync_copy(rows_v, out_hbm.at[pl.ds(out_base + j * _CH, _CH)])
        return carry

    lax.fori_loop(0, _CHUNKS, body, 0)


@jax.jit
def _gather(params, idx2d):
    mesh = plsc.VectorSubcoreMesh(core_axis_name="c", subcore_axis_name="s")
    return pl.kernel(
        _gather_body,
        out_type=jax.ShapeDtypeStruct((_B, _D), jnp.float32),
        mesh=mesh,
        scratch_types=[
            pltpu.VMEM((_CHUNKS, _CH), jnp.int32),
            pltpu.VMEM((_CH, _D), jnp.float32),
            pltpu.SemaphoreType.DMA,
        ],
        compiler_params=pltpu.CompilerParams(use_tc_tiling_on_sc=False),
    )(params, idx2d)


def kernel(params, indices):
    idx2d = indices.reshape(-1).astype(jnp.int32).reshape(_NW * _CHUNKS, _CH)
    out = _gather(params, idx2d)
    return out.reshape(indices.shape + (_D,))


# speedup vs baseline: 1.5690x; 1.0915x over previous
import jax
import jax.numpy as jnp
from jax import lax
from jax.experimental import pallas as pl
from jax.experimental.pallas import tpu as pltpu
from jax.experimental.pallas import tpu_sc as plsc

_V = 1_000_000
_D = 32
_B = 16384 * 26

_NC = 2
_NS = 16
_NW = _NC * _NS
_CH = 128
_B_PER_W = _B // _NW
_CHUNKS = _B_PER_W // _CH


_K = 4
_SUB = _K * _CH
_PAIRS = _B_PER_W // (2 * _SUB)


def _gather_body(table_hbm, idx_hbm, out_hbm, idx_v, rows_a, rows_b,
                 gsem_a, gsem_b, wsem_a, wsem_b):
    wid = lax.axis_index("s") * _NC + lax.axis_index("c")
    pltpu.sync_copy(idx_hbm.at[pl.ds(wid * _CHUNKS, _CHUNKS)], idx_v)
    out_base = wid * _B_PER_W

    def body(m, carry):
        c0 = 2 * _K * m
        ga = [
            pltpu.async_copy(table_hbm.at[idx_v.at[c0 + j]],
                             rows_a.at[pl.ds(j * _CH, _CH)], gsem_a)
            for j in range(_K)
        ]
        gb = [
            pltpu.async_copy(table_hbm.at[idx_v.at[c0 + _K + j]],
                             rows_b.at[pl.ds(j * _CH, _CH)], gsem_b)
            for j in range(_K)
        ]
        for d in ga:
            d.wait()
        wa = pltpu.async_copy(
            rows_a, out_hbm.at[pl.ds(out_base + 2 * _SUB * m, _SUB)], wsem_a)
        for d in gb:
            d.wait()
        wb = pltpu.async_copy(
            rows_b, out_hbm.at[pl.ds(out_base + 2 * _SUB * m + _SUB, _SUB)],
            wsem_b)
        wa.wait()
        wb.wait()
        return carry

    lax.fori_loop(0, _PAIRS, body, 0)


@jax.jit
def _gather(params, idx2d):
    mesh = plsc.VectorSubcoreMesh(core_axis_name="c", subcore_axis_name="s")
    return pl.kernel(
        _gather_body,
        out_type=jax.ShapeDtypeStruct((_B, _D), jnp.float32),
        mesh=mesh,
        scratch_types=[
            pltpu.VMEM((_CHUNKS, _CH), jnp.int32),
            pltpu.VMEM((_SUB, _D), jnp.float32),
            pltpu.VMEM((_SUB, _D), jnp.float32),
            pltpu.SemaphoreType.DMA,
            pltpu.SemaphoreType.DMA,
            pltpu.SemaphoreType.DMA,
            pltpu.SemaphoreType.DMA,
        ],
        compiler_params=pltpu.CompilerParams(use_tc_tiling_on_sc=False),
    )(params, idx2d)


def kernel(params, indices):
    idx2d = indices.reshape(-1).astype(jnp.int32).reshape(_NW * _CHUNKS, _CH)
    out = _gather(params, idx2d)
    return out.reshape(indices.shape + (_D,))


# speedup vs baseline: 1.8674x; 1.1901x over previous
import jax
import jax.numpy as jnp
from jax import lax
from jax.experimental import pallas as pl
from jax.experimental.pallas import tpu as pltpu
from jax.experimental.pallas import tpu_sc as plsc

_V = 1_000_000
_D = 32
_NI = 16384
_NJ = 26
_B = _NI * _NJ

_NC = 2
_NS = 16
_NW = _NC * _NS
_I_PER_W = _NI // _NW
_IC = 4
_CH = _IC * _NJ
_CHUNKS = _I_PER_W // _IC


def _gather_body(table_hbm, idx_hbm, out_hbm, idx_v, rows_a, rows_b,
                 gsem_a, gsem_b, wsem_a, wsem_b):
    wid = lax.axis_index("s") * _NC + lax.axis_index("c")
    pltpu.sync_copy(idx_hbm.at[pl.ds(wid * _CHUNKS, _CHUNKS)], idx_v)
    i_base = wid * _I_PER_W

    def write_out(rows_v, c, wsem):
        i0 = i_base + c * _IC
        return [
            pltpu.async_copy(
                rows_v.at[pl.ds(k * _NJ, _NJ)],
                out_hbm.at[i0 + k, pl.ds(0, _NJ), pl.ds(0, _D)],
                wsem)
            for k in range(_IC)
        ]

    def body(m, carry):
        c0 = 2 * m
        ga = pltpu.async_copy(table_hbm.at[idx_v.at[c0]], rows_a, gsem_a)
        gb = pltpu.async_copy(table_hbm.at[idx_v.at[c0 + 1]], rows_b, gsem_b)
        ga.wait()
        wa = write_out(rows_a, c0, wsem_a)
        gb.wait()
        wb = write_out(rows_b, c0 + 1, wsem_b)
        for d in wa:
            d.wait()
        for d in wb:
            d.wait()
        return carry

    lax.fori_loop(0, _CHUNKS // 2, body, 0)


@jax.jit
def _gather(params, idx2d):
    mesh = plsc.VectorSubcoreMesh(core_axis_name="c", subcore_axis_name="s")
    return pl.kernel(
        _gather_body,
        out_type=jax.ShapeDtypeStruct((_NI, 32, 128), jnp.float32),
        mesh=mesh,
        scratch_types=[
            pltpu.VMEM((_NW * _CHUNKS // _NW, _CH), jnp.int32),
            pltpu.VMEM((_CH, _D), jnp.float32),
            pltpu.VMEM((_CH, _D), jnp.float32),
            pltpu.SemaphoreType.DMA,
            pltpu.SemaphoreType.DMA,
            pltpu.SemaphoreType.DMA,
            pltpu.SemaphoreType.DMA,
        ],
        compiler_params=pltpu.CompilerParams(use_tc_tiling_on_sc=False),
    )(params, idx2d)


def kernel(params, indices):
    idx2d = indices.reshape(-1).astype(jnp.int32).reshape(_NW * _CHUNKS, _CH)
    out_big = _gather(params, idx2d)
    return out_big[:, :_NJ, :_D]
